# baseline (speedup 1.0000x reference)
import functools

import jax
import jax.numpy as jnp
from jax import lax
from jax.experimental import pallas as pl
from jax.experimental.pallas import tpu as pltpu
from jax.experimental.pallas import tpu_sc as plsc

VOCAB = 1000
DIM = 32
ROWS = 16384 * 50
NC, NS = 2, 16
NW = NC * NS
ROWS_PER_W = ROWS // NW
CHUNK = 1280
MACROS = ROWS_PER_W // CHUNK


def _make_sc_gather():
    mesh = plsc.VectorSubcoreMesh(core_axis_name="c", subcore_axis_name="s")

    @functools.partial(
        pl.kernel,
        mesh=mesh,
        compiler_params=pltpu.CompilerParams(use_tc_tiling_on_sc=False),
        out_type=jax.ShapeDtypeStruct((ROWS, DIM), jnp.float32),
        scratch_types=[
            pltpu.VMEM((ROWS_PER_W,), jnp.int32),
            pltpu.VMEM((CHUNK, DIM), jnp.float32),
            pltpu.VMEM((CHUNK, DIM), jnp.float32),
            pltpu.SemaphoreType.DMA,
            pltpu.SemaphoreType.DMA,
        ],
    )
    def k(table_hbm, idx_hbm, out_hbm, idx_v, rows0_v, rows1_v, sem_g, sem_o):
        wid = lax.axis_index("s") * NC + lax.axis_index("c")
        row0 = wid * ROWS_PER_W
        pltpu.sync_copy(idx_hbm.at[pl.ds(row0, ROWS_PER_W)], idx_v)

        def gather_into(m, buf):
            pltpu.async_copy(
                table_hbm.at[idx_v.at[pl.ds(m * CHUNK, CHUNK)]], buf, sem_g
            ).wait()

        def start_write(m, buf):
            pltpu.async_copy(
                buf, out_hbm.at[pl.ds(row0 + m * CHUNK, CHUNK)], sem_o
            )

        def wait_write(buf):
            pltpu.make_async_copy(
                buf, out_hbm.at[pl.ds(row0, CHUNK)], sem_o
            ).wait()

        gather_into(0, rows0_v)
        start_write(0, rows0_v)
        gather_into(1, rows1_v)
        start_write(1, rows1_v)

        def macro(mm, carry):
            m = 2 * mm
            wait_write(rows0_v)
            gather_into(m, rows0_v)
            start_write(m, rows0_v)
            wait_write(rows1_v)
            gather_into(m + 1, rows1_v)
            start_write(m + 1, rows1_v)
            return carry

        lax.fori_loop(1, MACROS // 2, macro, 0, unroll=False)

        wait_write(rows0_v)
        wait_write(rows1_v)

    return k


_sc_gather = _make_sc_gather()


def kernel(batch, weight):
    b, h = batch.shape
    idx = batch.reshape(ROWS).astype(jnp.int32)
    flat = _sc_gather(weight, idx)
    return flat.reshape(b, h, DIM)


# speedup vs baseline: 1.9863x; 1.9863x over previous
import functools

import jax
import jax.numpy as jnp
from jax import lax
from jax.experimental import pallas as pl
from jax.experimental.pallas import tpu as pltpu
from jax.experimental.pallas import tpu_sc as plsc

VOCAB = 1000
DIM = 32
B = 16384
H = 50
NC, NS = 2, 16
NW = NC * NS
B_PER_W = B // NW
NB = 16
MACROS = B_PER_W // NB


def _make_sc_gather():
    mesh = plsc.VectorSubcoreMesh(core_axis_name="c", subcore_axis_name="s")

    @functools.partial(
        pl.kernel,
        mesh=mesh,
        compiler_params=pltpu.CompilerParams(use_tc_tiling_on_sc=False),
        out_type=jax.ShapeDtypeStruct((B, H, DIM), jnp.float32),
        scratch_types=[
            pltpu.VMEM((B_PER_W, H), jnp.int32),
            pltpu.VMEM((NB, H, DIM), jnp.float32),
            pltpu.VMEM((NB, H, DIM), jnp.float32),
            pltpu.SemaphoreType.DMA,
            pltpu.SemaphoreType.DMA,
        ],
    )
    def k(table_hbm, idx_hbm, out_hbm, idx_v, rows0_v, rows1_v, sem_g, sem_o):
        wid = lax.axis_index("s") * NC + lax.axis_index("c")
        b0 = wid * B_PER_W
        pltpu.sync_copy(idx_hbm.at[pl.ds(b0, B_PER_W)], idx_v)

        def gather_into(m, buf):
            r0 = m * NB
            cps = [
                pltpu.async_copy(
                    table_hbm.at[idx_v.at[r0 + i]], buf.at[i], sem_g
                )
                for i in range(NB)
            ]
            for cp in cps:
                cp.wait()

        def start_write(m, buf):
            pltpu.async_copy(
                buf, out_hbm.at[pl.ds(b0 + m * NB, NB)], sem_o
            )

        def wait_write(buf):
            pltpu.make_async_copy(
                buf, out_hbm.at[pl.ds(b0, NB)], sem_o
            ).wait()

        gather_into(0, rows0_v)
        start_write(0, rows0_v)
        gather_into(1, rows1_v)
        start_write(1, rows1_v)

        def macro(mm, carry):
            m = 2 * mm
            wait_write(rows0_v)
            gather_into(m, rows0_v)
            start_write(m, rows0_v)
            wait_write(rows1_v)
            gather_into(m + 1, rows1_v)
            start_write(m + 1, rows1_v)
            return carry

        lax.fori_loop(1, MACROS // 2, macro, 0, unroll=False)

        wait_write(rows0_v)
        wait_write(rows1_v)

    return k


_sc_gather = _make_sc_gather()


def kernel(batch, weight):
    return _sc_gather(weight, batch.astype(jnp.int32))


# speedup vs baseline: 1.9874x; 1.0005x over previous
import functools

import jax
import jax.numpy as jnp
from jax import lax
from jax.experimental import pallas as pl
from jax.experimental.pallas import tpu as pltpu
from jax.experimental.pallas import tpu_sc as plsc

VOCAB = 1000
DIM = 32
B = 16384
H = 50
NC, NS = 2, 16
NW = NC * NS
B_PER_W = B // NW
NB = 16
MACROS = B_PER_W // NB


def _make_sc_gather():
    mesh = plsc.VectorSubcoreMesh(core_axis_name="c", subcore_axis_name="s")

    @functools.partial(
        pl.kernel,
        mesh=mesh,
        compiler_params=pltpu.CompilerParams(use_tc_tiling_on_sc=False),
        out_type=jax.ShapeDtypeStruct((B, H, DIM), jnp.float32),
        scratch_types=[
            pltpu.VMEM((B_PER_W, H), jnp.int32),
            pltpu.VMEM((NB, H, DIM), jnp.float32),
            pltpu.VMEM((NB, H, DIM), jnp.float32),
            pltpu.SemaphoreType.DMA,
            pltpu.SemaphoreType.DMA,
            pltpu.SemaphoreType.DMA,
        ],
    )
    def k(table_hbm, idx_hbm, out_hbm, idx_v, buf0, buf1, sg0, sg1, sem_o):
        wid = lax.axis_index("s") * NC + lax.axis_index("c")
        b0 = wid * B_PER_W
        pltpu.sync_copy(idx_hbm.at[pl.ds(b0, B_PER_W)], idx_v)

        def issue(m, buf, sem):
            r0 = m * NB
            for i in range(NB):
                pltpu.async_copy(table_hbm.at[idx_v.at[r0 + i]], buf.at[i], sem)

        def drain(buf, sem):
            for i in range(NB):
                pltpu.make_async_copy(
                    table_hbm.at[idx_v.at[i]], buf.at[i], sem
                ).wait()

        def start_write(m, buf):
            pltpu.async_copy(buf, out_hbm.at[pl.ds(b0 + m * NB, NB)], sem_o)

        def wait_write(buf):
            pltpu.make_async_copy(
                buf, out_hbm.at[pl.ds(b0, NB)], sem_o
            ).wait()

        issue(0, buf0, sg0)
        issue(1, buf1, sg1)
        drain(buf0, sg0)
        start_write(0, buf0)

        def body(mm, carry):
            m = 2 * mm + 1
            wait_write(buf0)
            issue(m + 1, buf0, sg0)
            drain(buf1, sg1)
            start_write(m, buf1)
            wait_write(buf1)
            issue(m + 2, buf1, sg1)
            drain(buf0, sg0)
            start_write(m + 1, buf0)
            return carry

        lax.fori_loop(0, (MACROS - 2) // 2, body, 0, unroll=False)

        drain(buf1, sg1)
        start_write(MACROS - 1, buf1)
        wait_write(buf0)
        wait_write(buf1)

    return k


_sc_gather = _make_sc_gather()


def kernel(batch, weight):
    return _sc_gather(weight, batch.astype(jnp.int32))
